# baseline (speedup 1.0000x reference)
import functools

import jax
import jax.numpy as jnp
from jax import lax
from jax.experimental import pallas as pl
from jax.experimental.pallas import tpu as pltpu
from jax.experimental.pallas import tpu_sc as plsc

N_NODES = 10000
N_EDGES = 320000
D_IN = 128
D_HID = 256
D_OUT = 128

NC = 2
NS = 16
NW = NC * NS
EB = 128
K_BATCH = 80
K_RING = K_BATCH + 2
E_PAD = NW * K_BATCH * EB
EBD = EB
KD = K_BATCH
E_PADD = E_PAD
N_PAD = 10112
ROWS_PER_TILE = N_PAD // NS
N_DEGPAD = 10240
DEG_PER_TILE = N_DEGPAD // NS


def _mesh():
    return plsc.VectorSubcoreMesh(
        core_axis_name="c", subcore_axis_name="s", num_cores=NC, num_subcores=NS)


def _sc_degree_body(dst_hbm, zeros_hbm, out_hbm, dst_v, ones_v, accum):
    cid = lax.axis_index("c")
    sid = lax.axis_index("s")
    wid = cid * NS + sid
    pltpu.sync_copy(dst_hbm.at[wid], dst_v)
    for j in range(EBD // 16):
        ones_v[pl.ds(j * 16, 16)] = jnp.ones((16,), jnp.float32)
    pltpu.sync_copy(zeros_hbm, accum.at[pl.ds(sid * DEG_PER_TILE, DEG_PER_TILE)])
    plsc.subcore_barrier()

    def body(k, carry):
        pltpu.sync_copy(ones_v, accum.at[dst_v.at[k]], add=True)
        return carry

    lax.fori_loop(0, KD, body, 0)
    plsc.subcore_barrier()
    pltpu.sync_copy(
        accum.at[pl.ds(sid * DEG_PER_TILE, DEG_PER_TILE)],
        out_hbm.at[pl.ds(cid * N_DEGPAD + sid * DEG_PER_TILE, DEG_PER_TILE)],
    )


@functools.cache
def _sc_degree_kernel():
    return pl.kernel(
        _sc_degree_body,
        out_type=jax.ShapeDtypeStruct((NC * N_DEGPAD,), jnp.float32),
        mesh=_mesh(),
        scratch_types=[
            pltpu.VMEM((KD, EBD), jnp.int32),
            pltpu.VMEM((EBD,), jnp.float32),
            pltpu.VMEM_SHARED((N_DEGPAD,), jnp.float32),
        ],
    )


def _sc_degree(dst3, zeros_deg):
    return _sc_degree_kernel()(dst3, zeros_deg)


def _sc_segsum_body(y_hbm, src_hbm, dst_hbm, zeros_hbm, out_hbm,
                    src_v, dstb0, dstb1, buf0, buf1, accum, sem, sems, semd):
    cid = lax.axis_index("c")
    sid = lax.axis_index("s")
    wid = cid * NS + sid
    pltpu.sync_copy(src_hbm.at[wid], src_v)
    pltpu.sync_copy(zeros_hbm, accum.at[pl.ds(sid * ROWS_PER_TILE, ROWS_PER_TILE)])
    plsc.subcore_barrier()

    pltpu.async_copy(y_hbm.at[src_v.at[0]], buf0, sem)
    pltpu.async_copy(y_hbm.at[src_v.at[1]], buf1, sem)
    pltpu.async_copy(dst_hbm.at[wid, 0], dstb0, semd)
    pltpu.async_copy(dst_hbm.at[wid, 1], dstb1, semd)

    def body(g, carry):
        k0 = 2 * g
        for b, (buf, dstb) in enumerate(((buf0, dstb0), (buf1, dstb1))):
            k = k0 + b
            pltpu.make_async_copy(y_hbm.at[src_v.at[0]], buf, sem).wait()
            pltpu.make_async_copy(dst_hbm.at[wid, 0], dstb, semd).wait()
            s = pltpu.async_copy(buf, accum.at[dstb], sems, add=True)
            s.wait()
            pltpu.async_copy(y_hbm.at[src_v.at[k + 2]], buf, sem)
            pltpu.async_copy(dst_hbm.at[wid, k + 2], dstb, semd)
        return carry

    lax.fori_loop(0, K_BATCH // 2, body, 0)
    pltpu.make_async_copy(y_hbm.at[src_v.at[0]], buf0, sem).wait()
    pltpu.make_async_copy(y_hbm.at[src_v.at[0]], buf1, sem).wait()
    pltpu.make_async_copy(dst_hbm.at[wid, 0], dstb0, semd).wait()
    pltpu.make_async_copy(dst_hbm.at[wid, 0], dstb1, semd).wait()
    plsc.subcore_barrier()
    pltpu.sync_copy(
        accum.at[pl.ds(sid * ROWS_PER_TILE, ROWS_PER_TILE)],
        out_hbm.at[cid, pl.ds(sid * ROWS_PER_TILE, ROWS_PER_TILE)],
    )


@functools.cache
def _sc_segsum_kernel():
    return pl.kernel(
        _sc_segsum_body,
        out_type=jax.ShapeDtypeStruct((NC, N_PAD, 128), jnp.float32),
        mesh=_mesh(),
        scratch_types=[
            pltpu.VMEM((K_RING, EB), jnp.int32),
            pltpu.VMEM((EB,), jnp.int32),
            pltpu.VMEM((EB,), jnp.int32),
            pltpu.VMEM((EB, 128), jnp.float32),
            pltpu.VMEM((EB, 128), jnp.float32),
            pltpu.VMEM_SHARED((N_PAD, 128), jnp.float32),
            pltpu.SemaphoreType.DMA,
            pltpu.SemaphoreType.DMA,
            pltpu.SemaphoreType.DMA,
        ],
    )


def _sc_segsum(y, src3, dst3, zeros_feat):
    return _sc_segsum_kernel()(y, src3, dst3, zeros_feat)


_R = 1000


def _dinv_block(deg_ref):
    d = deg_ref[0] + deg_ref[1] + 1.0
    return 1.0 / jnp.sqrt(d)


def _tc_pre_body(x_ref, w_ref, deg_ref, y_ref):
    dinv = _dinv_block(deg_ref)
    y_ref[0] = jnp.dot(x_ref[...], w_ref[...],
                       preferred_element_type=jnp.float32) * dinv


def _tc_pre(x, W1, deg3):
    return pl.pallas_call(
        _tc_pre_body,
        grid=(N_NODES // _R, D_HID // 128),
        in_specs=[
            pl.BlockSpec((_R, D_IN), lambda i, j: (i, 0)),
            pl.BlockSpec((D_IN, 128), lambda i, j: (0, j)),
            pl.BlockSpec((NC, _R, 1), lambda i, j: (0, i, 0)),
        ],
        out_specs=pl.BlockSpec((1, _R, 128), lambda i, j: (j, i, 0)),
        out_shape=jax.ShapeDtypeStruct((D_HID // 128, N_NODES, 128), jnp.float32),
    )(x, W1, deg3)


def _tc_mid_body(s0_ref, s1_ref, y1_ref, deg_ref, b1_ref, w2_ref, y2_ref):
    dinv = _dinv_block(deg_ref)
    ha = jax.nn.relu((s0_ref[0] + s0_ref[1] + y1_ref[0]) * dinv + b1_ref[0, :128][None, :])
    hb = jax.nn.relu((s1_ref[0] + s1_ref[1] + y1_ref[1]) * dinv + b1_ref[0, 128:][None, :])
    y2 = (jnp.dot(ha, w2_ref[:128, :], preferred_element_type=jnp.float32)
          + jnp.dot(hb, w2_ref[128:, :], preferred_element_type=jnp.float32))
    y2_ref[...] = y2 * dinv


def _tc_mid(s1a, s1b, y1, deg3, b1, W2):
    return pl.pallas_call(
        _tc_mid_body,
        grid=(N_NODES // _R,),
        in_specs=[
            pl.BlockSpec((NC, _R, 128), lambda i: (0, i, 0)),
            pl.BlockSpec((NC, _R, 128), lambda i: (0, i, 0)),
            pl.BlockSpec((2, _R, 128), lambda i: (0, i, 0)),
            pl.BlockSpec((NC, _R, 1), lambda i: (0, i, 0)),
            pl.BlockSpec((1, D_HID), lambda i: (0, 0)),
            pl.BlockSpec((D_HID, D_OUT), lambda i: (0, 0)),
        ],
        out_specs=pl.BlockSpec((_R, D_OUT), lambda i: (i, 0)),
        out_shape=jax.ShapeDtypeStruct((N_NODES, D_OUT), jnp.float32),
    )(s1a, s1b, y1, deg3, b1, W2)


def _tc_post_body(s_ref, y2_ref, deg_ref, b2_ref, out_ref):
    dinv = _dinv_block(deg_ref)
    z = (s_ref[0] + s_ref[1] + y2_ref[...]) * dinv + b2_ref[0][None, :]
    m = jnp.max(z, axis=1, keepdims=True)
    e = jnp.exp(z - m)
    out_ref[...] = z - m - jnp.log(jnp.sum(e, axis=1, keepdims=True))


def _tc_post(s2, y2, deg3, b2):
    return pl.pallas_call(
        _tc_post_body,
        grid=(N_NODES // _R,),
        in_specs=[
            pl.BlockSpec((NC, _R, 128), lambda i: (0, i, 0)),
            pl.BlockSpec((_R, D_OUT), lambda i: (i, 0)),
            pl.BlockSpec((NC, _R, 1), lambda i: (0, i, 0)),
            pl.BlockSpec((1, D_OUT), lambda i: (0, 0)),
        ],
        out_specs=pl.BlockSpec((_R, D_OUT), lambda i: (i, 0)),
        out_shape=jax.ShapeDtypeStruct((N_NODES, D_OUT), jnp.float32),
    )(s2, y2, deg3, b2)


def kernel(x, adjacency_matrix, W1, b1, W2, b2):
    src = adjacency_matrix[0].astype(jnp.int32)
    dst = adjacency_matrix[1].astype(jnp.int32)
    pad = E_PAD - N_EDGES
    src3 = jnp.concatenate([src, jnp.zeros((pad,), jnp.int32)]).reshape(NW, K_BATCH, EB)
    src3 = jnp.concatenate([src3, jnp.zeros((NW, K_RING - K_BATCH, EB), jnp.int32)], axis=1)
    dst3d = jnp.concatenate([dst, jnp.full((pad,), N_NODES, jnp.int32)]).reshape(NW, K_BATCH, EB)
    dst3 = jnp.concatenate(
        [dst3d, jnp.full((NW, K_RING - K_BATCH, EB), N_NODES, jnp.int32)], axis=1)
    zeros_deg = jnp.zeros((DEG_PER_TILE,), jnp.float32)
    zeros_feat = jnp.zeros((ROWS_PER_TILE, 128), jnp.float32)
    b1r = b1.reshape(1, D_HID)
    b2r = b2.reshape(1, D_OUT)

    deg = _sc_degree(dst3d, zeros_deg)
    deg3 = deg.reshape(NC, N_DEGPAD, 1)

    y1 = _tc_pre(x, W1, deg3)
    s1a = _sc_segsum(y1[0], src3, dst3, zeros_feat)
    s1b = _sc_segsum(y1[1], src3, dst3, zeros_feat)

    y2 = _tc_mid(s1a, s1b, y1, deg3, b1r, W2)
    s2 = _sc_segsum(y2, src3, dst3, zeros_feat)

    return _tc_post(s2, y2, deg3, b2r)


# speedup vs baseline: 2.3485x; 2.3485x over previous
import functools

import jax
import jax.numpy as jnp
from jax import lax
from jax.experimental import pallas as pl
from jax.experimental.pallas import tpu as pltpu
from jax.experimental.pallas import tpu_sc as plsc

N_NODES = 10000
N_EDGES = 320000
D_IN = 128
D_HID = 256
D_OUT = 128

NC = 2
NS = 16
NW = NC * NS
EB = 128
K_BATCH = -(-N_EDGES // (NW * EB))
E_PAD = NW * K_BATCH * EB
N_PAD = 10112
ROWS_PER_TILE = N_PAD // NS
N_DEGPAD = 10240
DEG_PER_TILE = N_DEGPAD // NS


def _mesh():
    return plsc.VectorSubcoreMesh(
        core_axis_name="c", subcore_axis_name="s", num_cores=NC, num_subcores=NS)


def _sc_degree_body(dst_hbm, zeros_hbm, out_hbm, dst_v, ones_v, accum):
    cid = lax.axis_index("c")
    sid = lax.axis_index("s")
    wid = cid * NS + sid
    pltpu.sync_copy(dst_hbm.at[wid], dst_v)
    for j in range(EB // 16):
        ones_v[pl.ds(j * 16, 16)] = jnp.ones((16,), jnp.float32)
    pltpu.sync_copy(zeros_hbm, accum.at[pl.ds(sid * DEG_PER_TILE, DEG_PER_TILE)])
    plsc.subcore_barrier()

    def body(k, carry):
        pltpu.sync_copy(ones_v, accum.at[dst_v.at[k]], add=True)
        return carry

    lax.fori_loop(0, K_BATCH, body, 0)
    plsc.subcore_barrier()
    pltpu.sync_copy(
        accum.at[pl.ds(sid * DEG_PER_TILE, DEG_PER_TILE)],
        out_hbm.at[pl.ds(cid * N_DEGPAD + sid * DEG_PER_TILE, DEG_PER_TILE)],
    )


@functools.cache
def _sc_degree_kernel():
    return pl.kernel(
        _sc_degree_body,
        out_type=jax.ShapeDtypeStruct((NC * N_DEGPAD,), jnp.float32),
        mesh=_mesh(),
        scratch_types=[
            pltpu.VMEM((K_BATCH, EB), jnp.int32),
            pltpu.VMEM((EB,), jnp.float32),
            pltpu.VMEM_SHARED((N_DEGPAD,), jnp.float32),
        ],
    )


def _sc_degree(dst3, zeros_deg):
    return _sc_degree_kernel()(dst3, zeros_deg)


def _sc_segsum_body(y_hbm, src_hbm, dst_hbm, zeros_hbm, out_hbm,
                    src_v, dst_v, buf, accum, sem):
    cid = lax.axis_index("c")
    sid = lax.axis_index("s")
    wid = cid * NS + sid
    pltpu.sync_copy(src_hbm.at[wid], src_v)
    pltpu.sync_copy(dst_hbm.at[wid], dst_v)
    pltpu.sync_copy(zeros_hbm, accum.at[pl.ds(sid * ROWS_PER_TILE, ROWS_PER_TILE)])
    plsc.subcore_barrier()

    def body(k, carry):
        pltpu.async_copy(y_hbm.at[src_v.at[k]], buf, sem).wait()
        pltpu.sync_copy(buf, accum.at[dst_v.at[k]], add=True)
        return carry

    lax.fori_loop(0, K_BATCH, body, 0)
    plsc.subcore_barrier()
    pltpu.sync_copy(
        accum.at[pl.ds(sid * ROWS_PER_TILE, ROWS_PER_TILE)],
        out_hbm.at[cid, pl.ds(sid * ROWS_PER_TILE, ROWS_PER_TILE)],
    )


@functools.cache
def _sc_segsum_kernel():
    return pl.kernel(
        _sc_segsum_body,
        out_type=jax.ShapeDtypeStruct((NC, N_PAD, 128), jnp.float32),
        mesh=_mesh(),
        scratch_types=[
            pltpu.VMEM((K_BATCH, EB), jnp.int32),
            pltpu.VMEM((K_BATCH, EB), jnp.int32),
            pltpu.VMEM((EB, 128), jnp.float32),
            pltpu.VMEM_SHARED((N_PAD, 128), jnp.float32),
            pltpu.SemaphoreType.DMA,
        ],
    )


def _sc_segsum(y, src3, dst3, zeros_feat):
    return _sc_segsum_kernel()(y, src3, dst3, zeros_feat)


_R = 1000


def _dinv_block(deg_ref):
    d = deg_ref[0] + deg_ref[1] + 1.0
    return 1.0 / jnp.sqrt(d)


def _tc_pre_body(x_ref, w_ref, deg_ref, y_ref):
    dinv = _dinv_block(deg_ref)
    y_ref[0] = jnp.dot(x_ref[...], w_ref[...],
                       preferred_element_type=jnp.float32) * dinv


def _tc_pre(x, W1, deg3):
    return pl.pallas_call(
        _tc_pre_body,
        grid=(N_NODES // _R, D_HID // 128),
        in_specs=[
            pl.BlockSpec((_R, D_IN), lambda i, j: (i, 0)),
            pl.BlockSpec((D_IN, 128), lambda i, j: (0, j)),
            pl.BlockSpec((NC, _R, 1), lambda i, j: (0, i, 0)),
        ],
        out_specs=pl.BlockSpec((1, _R, 128), lambda i, j: (j, i, 0)),
        out_shape=jax.ShapeDtypeStruct((D_HID // 128, N_NODES, 128), jnp.float32),
    )(x, W1, deg3)


def _tc_mid_body(s0_ref, s1_ref, y1_ref, deg_ref, b1_ref, w2_ref, y2_ref):
    dinv = _dinv_block(deg_ref)
    ha = jax.nn.relu((s0_ref[0] + s0_ref[1] + y1_ref[0]) * dinv + b1_ref[0, :128][None, :])
    hb = jax.nn.relu((s1_ref[0] + s1_ref[1] + y1_ref[1]) * dinv + b1_ref[0, 128:][None, :])
    y2 = (jnp.dot(ha, w2_ref[:128, :], preferred_element_type=jnp.float32)
          + jnp.dot(hb, w2_ref[128:, :], preferred_element_type=jnp.float32))
    y2_ref[...] = y2 * dinv


def _tc_mid(s1a, s1b, y1, deg3, b1, W2):
    return pl.pallas_call(
        _tc_mid_body,
        grid=(N_NODES // _R,),
        in_specs=[
            pl.BlockSpec((NC, _R, 128), lambda i: (0, i, 0)),
            pl.BlockSpec((NC, _R, 128), lambda i: (0, i, 0)),
            pl.BlockSpec((2, _R, 128), lambda i: (0, i, 0)),
            pl.BlockSpec((NC, _R, 1), lambda i: (0, i, 0)),
            pl.BlockSpec((1, D_HID), lambda i: (0, 0)),
            pl.BlockSpec((D_HID, D_OUT), lambda i: (0, 0)),
        ],
        out_specs=pl.BlockSpec((_R, D_OUT), lambda i: (i, 0)),
        out_shape=jax.ShapeDtypeStruct((N_NODES, D_OUT), jnp.float32),
    )(s1a, s1b, y1, deg3, b1, W2)


def _tc_post_body(s_ref, y2_ref, deg_ref, b2_ref, out_ref):
    dinv = _dinv_block(deg_ref)
    z = (s_ref[0] + s_ref[1] + y2_ref[...]) * dinv + b2_ref[0][None, :]
    m = jnp.max(z, axis=1, keepdims=True)
    e = jnp.exp(z - m)
    out_ref[...] = z - m - jnp.log(jnp.sum(e, axis=1, keepdims=True))


def _tc_post(s2, y2, deg3, b2):
    return pl.pallas_call(
        _tc_post_body,
        grid=(N_NODES // _R,),
        in_specs=[
            pl.BlockSpec((NC, _R, 128), lambda i: (0, i, 0)),
            pl.BlockSpec((_R, D_OUT), lambda i: (i, 0)),
            pl.BlockSpec((NC, _R, 1), lambda i: (0, i, 0)),
            pl.BlockSpec((1, D_OUT), lambda i: (0, 0)),
        ],
        out_specs=pl.BlockSpec((_R, D_OUT), lambda i: (i, 0)),
        out_shape=jax.ShapeDtypeStruct((N_NODES, D_OUT), jnp.float32),
    )(s2, y2, deg3, b2)


def kernel(x, adjacency_matrix, W1, b1, W2, b2):
    src = adjacency_matrix[0].astype(jnp.int32)
    dst = adjacency_matrix[1].astype(jnp.int32)
    pad = E_PAD - N_EDGES
    src3 = jnp.concatenate([src, jnp.zeros((pad,), jnp.int32)]).reshape(NW, K_BATCH, EB)
    dst3 = jnp.concatenate([dst, jnp.full((pad,), N_NODES, jnp.int32)]).reshape(NW, K_BATCH, EB)
    zeros_deg = jnp.zeros((DEG_PER_TILE,), jnp.float32)
    zeros_feat = jnp.zeros((ROWS_PER_TILE, 128), jnp.float32)
    b1r = b1.reshape(1, D_HID)
    b2r = b2.reshape(1, D_OUT)

    deg = _sc_degree(dst3, zeros_deg)
    deg3 = deg.reshape(NC, N_DEGPAD, 1)

    y1 = _tc_pre(x, W1, deg3)
    s1a = _sc_segsum(y1[0], src3, dst3, zeros_feat)
    s1b = _sc_segsum(y1[1], src3, dst3, zeros_feat)

    y2 = _tc_mid(s1a, s1b, y1, deg3, b1r, W2)
    s2 = _sc_segsum(y2, src3, dst3, zeros_feat)

    return _tc_post(s2, y2, deg3, b2r)
